# baseline (speedup 1.0000x reference)
import functools

import jax
import jax.numpy as jnp
from jax import lax
from jax.experimental import pallas as pl
from jax.experimental.pallas import tpu as pltpu
from jax.experimental.pallas import tpu_sc as plsc

_N = 10000
_E = 320000
_D = 128
_CHUNK = 100
_CHUNKS = _E // _CHUNK
_STAGE = 32
_ROWS_PER_TILE = 632
_N_PAD = 10112


def _hop(src, rows_c, cols_c, zeros):
    mesh = plsc.VectorSubcoreMesh(core_axis_name="c", subcore_axis_name="s")

    @functools.partial(
        pl.kernel,
        out_type=jax.ShapeDtypeStruct((2, _N_PAD, _D), jnp.float32),
        mesh=mesh,
        scratch_types=[
            pltpu.VMEM((_STAGE, _CHUNK), jnp.int32),
            pltpu.VMEM((_STAGE, _CHUNK), jnp.int32),
            pltpu.VMEM((_CHUNK, _D), jnp.float32),
            pltpu.VMEM((_CHUNK, _D), jnp.float32),
            pltpu.VMEM((_CHUNK, _D), jnp.float32),
            pltpu.VMEM_SHARED((_N_PAD, _D), jnp.float32),
            pltpu.SemaphoreType.DMA,
            pltpu.SemaphoreType.DMA,
            pltpu.SemaphoreType.DMA,
            pltpu.SemaphoreType.DMA,
            pltpu.SemaphoreType.DMA,
            pltpu.SemaphoreType.DMA,
        ],
    )
    def hop_kernel(src_hbm, rows_hbm, cols_hbm, zeros_hbm, out_hbm,
                   col_v, row_v, g0, g1, g2, acc_sh,
                   gs0, gs1, gs2, ss0, ss1, ss2):
        c = lax.axis_index("c")
        s = lax.axis_index("s")
        wid = s * 2 + c

        pltpu.sync_copy(zeros_hbm, acc_sh.at[pl.ds(s * _ROWS_PER_TILE,
                                                   _ROWS_PER_TILE)])
        plsc.subcore_barrier()

        bufs = (g0, g1, g2)
        gsems = (gs0, gs1, gs2)
        ssems = (ss0, ss1, ss2)

        def gwait(k, b):
            pltpu.make_async_copy(src_hbm.at[col_v.at[k]],
                                  bufs[b], gsems[b]).wait()

        def swait(k, b):
            pltpu.make_async_copy(bufs[b], acc_sh.at[row_v.at[k]],
                                  ssems[b]).wait()

        def run_stage(base, size):
            pltpu.sync_copy(cols_hbm.at[pl.ds(base, size)],
                            col_v.at[pl.ds(0, size)])
            pltpu.sync_copy(rows_hbm.at[pl.ds(base, size)],
                            row_v.at[pl.ds(0, size)])

            pltpu.async_copy(src_hbm.at[col_v.at[0]], bufs[0], gsems[0])

            def stp(kk, b, first2=False, static_k=False):
                gwait(kk, b)
                pltpu.async_copy(bufs[b], acc_sh.at[row_v.at[kk]],
                                 ssems[b], add=True)
                bnext = (b + 1) % 3
                if not first2:
                    swait(kk, bnext)

                if static_k:
                    if kk + 1 < size:
                        pltpu.async_copy(src_hbm.at[col_v.at[kk + 1]],
                                         bufs[bnext], gsems[bnext])
                else:
                    @pl.when(kk + 1 < size)
                    def _():
                        pltpu.async_copy(src_hbm.at[col_v.at[kk + 1]],
                                         bufs[bnext], gsems[bnext])

            stp(0, 0, first2=True, static_k=True)
            stp(1, 1, first2=True, static_k=True)

            def body3(k3, carry):
                k = 3 * k3 + 2
                stp(k, 2)
                stp(k + 1, 0)
                stp(k + 2, 1)
                return carry

            lax.fori_loop(0, (size - 2) // 3, body3, 0)
            for kk in range(2 + 3 * ((size - 2) // 3), size):
                stp(kk, kk % 3, static_k=True)
            swait(size - 2, (size - 2) % 3)
            swait(size - 1, (size - 1) % 3)

        @pl.when(wid < 16)
        def _():
            base = wid * 104
            run_stage(base, 32)
            run_stage(base + 32, 32)
            run_stage(base + 64, 32)
            run_stage(base + 96, 8)

        @pl.when(wid >= 16)
        def _():
            base = 1664 + (wid - 16) * 96
            run_stage(base, 32)
            run_stage(base + 32, 32)
            run_stage(base + 64, 32)

        plsc.subcore_barrier()

        pltpu.sync_copy(acc_sh.at[pl.ds(s * _ROWS_PER_TILE, _ROWS_PER_TILE)],
                        out_hbm.at[c, pl.ds(s * _ROWS_PER_TILE,
                                            _ROWS_PER_TILE)])

    return hop_kernel(src, rows_c, cols_c, zeros)


def _combine_body(p0_ref, p1_ref, o_ref):
    o_ref[...] = p0_ref[0] + p1_ref[0]


def _combine(p):
    blk = 1264
    return pl.pallas_call(
        _combine_body,
        grid=(_N_PAD // blk,),
        in_specs=[
            pl.BlockSpec((1, blk, _D), lambda i: (0, i, 0)),
            pl.BlockSpec((1, blk, _D), lambda i: (1, i, 0)),
        ],
        out_specs=pl.BlockSpec((blk, _D), lambda i: (i, 0)),
        out_shape=jax.ShapeDtypeStruct((_N_PAD, _D), jnp.float32),
    )(p, p)


def _final_body(x_ref, x1_ref, q0_ref, q1_ref, w0_ref, w1_ref, w2_ref, b_ref,
                o_ref):
    x2 = q0_ref[0] + q1_ref[0]
    acc = jnp.dot(x_ref[...], w0_ref[...], preferred_element_type=jnp.float32)
    acc = acc + jnp.dot(x1_ref[...], w1_ref[...],
                        preferred_element_type=jnp.float32)
    acc = acc + jnp.dot(x2, w2_ref[...], preferred_element_type=jnp.float32)
    o_ref[...] = acc + b_ref[...]


def _final(x, x1, q, w0, w1, w2, b):
    blk = 1000
    return pl.pallas_call(
        _final_body,
        grid=(_N // blk,),
        in_specs=[
            pl.BlockSpec((blk, _D), lambda i: (i, 0)),
            pl.BlockSpec((blk, _D), lambda i: (i, 0)),
            pl.BlockSpec((1, blk, _D), lambda i: (0, i, 0)),
            pl.BlockSpec((1, blk, _D), lambda i: (1, i, 0)),
            pl.BlockSpec((_D, _D), lambda i: (0, 0)),
            pl.BlockSpec((_D, _D), lambda i: (0, 0)),
            pl.BlockSpec((_D, _D), lambda i: (0, 0)),
            pl.BlockSpec((1, _D), lambda i: (0, 0)),
        ],
        out_specs=pl.BlockSpec((blk, _D), lambda i: (i, 0)),
        out_shape=jax.ShapeDtypeStruct((_N, _D), jnp.float32),
    )(x, x1, q, q, w0, w1, w2, b)


def kernel(x, edge_index, batch, W0_0, W0_1, W0_2, b0, W1_0, W1_1, W1_2, b1):
    rows_c = edge_index[0].reshape(_CHUNKS, _CHUNK)
    cols_c = edge_index[1].reshape(_CHUNKS, _CHUNK)
    zeros = jnp.zeros((_ROWS_PER_TILE, _D), jnp.float32)

    p = _hop(x, rows_c, cols_c, zeros)
    x1 = _combine(p)
    q = _hop(x1, rows_c, cols_c, zeros)
    return _final(x, x1, q, W1_0, W1_1, W1_2, b1.reshape(1, _D))


# speedup vs baseline: 1.3695x; 1.3695x over previous
import functools

import jax
import jax.numpy as jnp
from jax import lax
from jax.experimental import pallas as pl
from jax.experimental.pallas import tpu as pltpu
from jax.experimental.pallas import tpu_sc as plsc

_N = 10000
_E = 320000
_D = 128
_CHUNK = 128
_CHUNKS = _E // _CHUNK
_STAGE = 40
_ROWS_PER_TILE = 632
_N_PAD = 10112


def _hop(src, ei, zeros):
    mesh = plsc.VectorSubcoreMesh(core_axis_name="c", subcore_axis_name="s")

    @functools.partial(
        pl.kernel,
        out_type=jax.ShapeDtypeStruct((2, _N_PAD, _D), jnp.float32),
        mesh=mesh,
        scratch_types=[
            pltpu.VMEM((_STAGE * _CHUNK,), jnp.int32),
            pltpu.VMEM((_STAGE * _CHUNK,), jnp.int32),
            pltpu.VMEM((_CHUNK, _D), jnp.float32),
            pltpu.VMEM((_CHUNK, _D), jnp.float32),
            pltpu.VMEM_SHARED((_N_PAD, _D), jnp.float32),
            pltpu.SemaphoreType.DMA,
            pltpu.SemaphoreType.DMA,
        ],
    )
    def hop_kernel(src_hbm, ei_hbm, zeros_hbm, out_hbm,
                   col_v, row_v, gath0_v, gath1_v, acc_sh, sem0, sem1):
        c = lax.axis_index("c")
        s = lax.axis_index("s")
        wid = s * 2 + c

        pltpu.sync_copy(zeros_hbm, acc_sh.at[pl.ds(s * _ROWS_PER_TILE,
                                                   _ROWS_PER_TILE)])
        plsc.subcore_barrier()

        bufs = (gath0_v, gath1_v)
        sems = (sem0, sem1)

        def run_stage(base, size):
            pltpu.sync_copy(ei_hbm.at[1, pl.ds(base * _CHUNK, size * _CHUNK)],
                            col_v.at[pl.ds(0, size * _CHUNK)])
            pltpu.sync_copy(ei_hbm.at[0, pl.ds(base * _CHUNK, size * _CHUNK)],
                            row_v.at[pl.ds(0, size * _CHUNK)])

            pltpu.async_copy(src_hbm.at[col_v.at[pl.ds(0, _CHUNK)]],
                             bufs[0], sems[0])

            def step(k, b, prefetch=True):
                if prefetch:
                    @pl.when(k + 1 < size)
                    def _():
                        pltpu.async_copy(
                            src_hbm.at[col_v.at[pl.ds((k + 1) * _CHUNK,
                                                      _CHUNK)]],
                            bufs[1 - b], sems[1 - b])
                pltpu.make_async_copy(
                    src_hbm.at[col_v.at[pl.ds(k * _CHUNK, _CHUNK)]],
                    bufs[b], sems[b]).wait()
                pltpu.sync_copy(
                    bufs[b], acc_sh.at[row_v.at[pl.ds(k * _CHUNK, _CHUNK)]],
                    add=True)

            def body2(k2, carry):
                k = 2 * k2
                step(k, 0)
                step(k + 1, 1)
                return carry

            lax.fori_loop(0, size // 2, body2, 0)

        @pl.when(wid < 31)
        def _():
            run_stage(wid * 80, 40)
            run_stage(wid * 80 + 40, 40)

        @pl.when(wid == 31)
        def _():
            run_stage(2480, 20)

        plsc.subcore_barrier()

        pltpu.sync_copy(acc_sh.at[pl.ds(s * _ROWS_PER_TILE, _ROWS_PER_TILE)],
                        out_hbm.at[c, pl.ds(s * _ROWS_PER_TILE,
                                            _ROWS_PER_TILE)])

    return hop_kernel(src, ei, zeros)


def _combine_body(p0_ref, p1_ref, o_ref):
    o_ref[...] = p0_ref[0] + p1_ref[0]


def _combine(p):
    blk = 1264
    return pl.pallas_call(
        _combine_body,
        grid=(_N_PAD // blk,),
        in_specs=[
            pl.BlockSpec((1, blk, _D), lambda i: (0, i, 0)),
            pl.BlockSpec((1, blk, _D), lambda i: (1, i, 0)),
        ],
        out_specs=pl.BlockSpec((blk, _D), lambda i: (i, 0)),
        out_shape=jax.ShapeDtypeStruct((_N_PAD, _D), jnp.float32),
    )(p, p)


def _final_body(x_ref, x1_ref, q0_ref, q1_ref, w0_ref, w1_ref, w2_ref, b_ref,
                o_ref):
    x2 = q0_ref[0] + q1_ref[0]
    acc = jnp.dot(x_ref[...], w0_ref[...], preferred_element_type=jnp.float32)
    acc = acc + jnp.dot(x1_ref[...], w1_ref[...],
                        preferred_element_type=jnp.float32)
    acc = acc + jnp.dot(x2, w2_ref[...], preferred_element_type=jnp.float32)
    o_ref[...] = acc + b_ref[...]


def _final(x, x1, q, w0, w1, w2, b):
    blk = 1000
    return pl.pallas_call(
        _final_body,
        grid=(_N // blk,),
        in_specs=[
            pl.BlockSpec((blk, _D), lambda i: (i, 0)),
            pl.BlockSpec((blk, _D), lambda i: (i, 0)),
            pl.BlockSpec((1, blk, _D), lambda i: (0, i, 0)),
            pl.BlockSpec((1, blk, _D), lambda i: (1, i, 0)),
            pl.BlockSpec((_D, _D), lambda i: (0, 0)),
            pl.BlockSpec((_D, _D), lambda i: (0, 0)),
            pl.BlockSpec((_D, _D), lambda i: (0, 0)),
            pl.BlockSpec((1, _D), lambda i: (0, 0)),
        ],
        out_specs=pl.BlockSpec((blk, _D), lambda i: (i, 0)),
        out_shape=jax.ShapeDtypeStruct((_N, _D), jnp.float32),
    )(x, x1, q, q, w0, w1, w2, b)


def kernel(x, edge_index, batch, W0_0, W0_1, W0_2, b0, W1_0, W1_1, W1_2, b1):
    zeros = jnp.zeros((_ROWS_PER_TILE, _D), jnp.float32)

    p = _hop(x, edge_index, zeros)
    x1 = _combine(p)
    q = _hop(x1, edge_index, zeros)
    return _final(x, x1, q, W1_0, W1_1, W1_2, b1.reshape(1, _D))
